# Initial kernel scaffold; baseline (speedup 1.0000x reference)
import jax
import jax.numpy as jnp
from jax.experimental import pallas as pl


def kernel(step, lprobs_ch0, lprobs_ch1, scores_ch0, scores_ch1):
    raise NotImplementedError("write your pallas kernel here")


# speedup vs baseline: 17.9109x; 17.9109x over previous
import jax
import jax.numpy as jnp
from jax.experimental import pallas as pl
from jax.experimental.pallas import tpu as pltpu

BSZ, BEAM, V = 32, 8, 32768
K = 2 * BEAM
ROWS, LANES = 256, 128
NEG = float("-inf")
BIG = 1 << 30


def _topk_per_beam(x, vidx, oh16):
    tv = jnp.zeros((BEAM, K), jnp.float32)
    ti = jnp.zeros((BEAM, K), jnp.int32)
    for t in range(K):
        m = jnp.max(x, axis=(1, 2), keepdims=True)
        cand = jnp.where(x == m, vidx, BIG)
        idx = jnp.min(cand, axis=(1, 2), keepdims=True)
        x = jnp.where(vidx == idx, NEG, x)
        tv = tv + jnp.where(oh16[t], m[:, :, 0], 0.0)
        ti = ti + jnp.where(oh16[t], idx[:, :, 0], 0)
    return tv, ti


def _body(lp0_ref, lp1_ref, sc0_ref, sc1_ref,
          s0_ref, s1_ref, t0_ref, t1_ref, ib_ref):
    x0 = lp0_ref[0] + sc0_ref[0]
    x1 = lp1_ref[0] + sc1_ref[0]

    vidx = (jax.lax.broadcasted_iota(jnp.int32, (BEAM, ROWS, LANES), 1) * LANES
            + jax.lax.broadcasted_iota(jnp.int32, (BEAM, ROWS, LANES), 2))
    lane16 = jax.lax.broadcasted_iota(jnp.int32, (1, K), 1)
    oh16 = [lane16 == t for t in range(K)]

    tv0, ti0 = _topk_per_beam(x0, vidx, oh16)
    tv1, ti1 = _topk_per_beam(x1, vidx, oh16)

    ss = tv0[:, :, None] + tv1[:, None, :]
    fidx = (jax.lax.broadcasted_iota(jnp.int32, (BEAM, K, K), 0) * (K * K)
            + jax.lax.broadcasted_iota(jnp.int32, (BEAM, K, K), 1) * K
            + jax.lax.broadcasted_iota(jnp.int32, (BEAM, K, K), 2))
    beam_i = jax.lax.broadcasted_iota(jnp.int32, (BEAM, K), 0)
    col_i = jax.lax.broadcasted_iota(jnp.int32, (BEAM, K), 1)

    s0a = jnp.zeros((1, K), jnp.float32)
    s1a = jnp.zeros((1, K), jnp.float32)
    t0a = jnp.zeros((1, K), jnp.int32)
    t1a = jnp.zeros((1, K), jnp.int32)
    iba = jnp.zeros((1, K), jnp.int32)
    for t in range(K):
        m = jnp.max(ss)
        idx = jnp.min(jnp.where(ss == m, fidx, BIG))
        ss = jnp.where(fidx == idx, NEG, ss)
        ib = idx >> 8
        rem = idx & 255
        i0 = rem >> 4
        i1 = rem & 15
        sel0 = (beam_i == ib) & (col_i == i0)
        sel1 = (beam_i == ib) & (col_i == i1)
        v0 = jnp.sum(jnp.where(sel0, tv0, 0.0))
        n0 = jnp.sum(jnp.where(sel0, ti0, 0))
        v1 = jnp.sum(jnp.where(sel1, tv1, 0.0))
        n1 = jnp.sum(jnp.where(sel1, ti1, 0))
        oh = oh16[t]
        s0a = s0a + jnp.where(oh, v0, 0.0)
        s1a = s1a + jnp.where(oh, v1, 0.0)
        t0a = t0a + jnp.where(oh, n0, 0)
        t1a = t1a + jnp.where(oh, n1, 0)
        iba = iba + jnp.where(oh, ib, 0)

    s0_ref[0] = s0a
    s1_ref[0] = s1a
    t0_ref[0] = t0a
    t1_ref[0] = t1a
    ib_ref[0] = iba


def kernel(step, lprobs_ch0, lprobs_ch1, scores_ch0, scores_ch1):
    sc0 = jax.lax.dynamic_index_in_dim(scores_ch0, step - 1, axis=2,
                                       keepdims=False)
    sc1 = jax.lax.dynamic_index_in_dim(scores_ch1, step - 1, axis=2,
                                       keepdims=False)
    lp0 = lprobs_ch0.reshape(BSZ, BEAM, ROWS, LANES)
    lp1 = lprobs_ch1.reshape(BSZ, BEAM, ROWS, LANES)
    sc0 = sc0.reshape(BSZ, BEAM, 1, 1)
    sc1 = sc1.reshape(BSZ, BEAM, 1, 1)

    out_shapes = (
        jax.ShapeDtypeStruct((BSZ, 1, K), jnp.float32),
        jax.ShapeDtypeStruct((BSZ, 1, K), jnp.float32),
        jax.ShapeDtypeStruct((BSZ, 1, K), jnp.int32),
        jax.ShapeDtypeStruct((BSZ, 1, K), jnp.int32),
        jax.ShapeDtypeStruct((BSZ, 1, K), jnp.int32),
    )
    in_specs = [
        pl.BlockSpec((1, BEAM, ROWS, LANES), lambda b: (b, 0, 0, 0)),
        pl.BlockSpec((1, BEAM, ROWS, LANES), lambda b: (b, 0, 0, 0)),
        pl.BlockSpec((1, BEAM, 1, 1), lambda b: (b, 0, 0, 0)),
        pl.BlockSpec((1, BEAM, 1, 1), lambda b: (b, 0, 0, 0)),
    ]
    out_specs = tuple(pl.BlockSpec((1, 1, K), lambda b: (b, 0, 0))
                      for _ in range(5))
    s0, s1, t0, t1, ib = pl.pallas_call(
        _body,
        grid=(BSZ,),
        in_specs=in_specs,
        out_specs=out_specs,
        out_shape=out_shapes,
        compiler_params=pltpu.CompilerParams(
            dimension_semantics=("arbitrary",),
        ),
    )(lp0, lp1, sc0, sc1)
    return (s0[:, 0, :], s1[:, 0, :], t0[:, 0, :], t1[:, 0, :], ib[:, 0, :])
